# baseline (speedup 1.0000x reference)
import functools

import jax
import jax.numpy as jnp
from jax import lax
from jax.experimental import pallas as pl
from jax.experimental.pallas import tpu as pltpu
from jax.experimental.pallas import tpu_sc as plsc

VOCAB = 1000000
HIDDEN = 64
EPS = 1e-5

NW = 32
KTOK = 512


def _rsqrt16(x):
    xi = plsc.bitcast(x, jnp.int32)
    yi = jnp.int32(0x5F3759DF) - (xi >> 1)
    y = plsc.bitcast(yi, jnp.float32)
    half_x = 0.5 * x
    for _ in range(3):
        y = y * (1.5 - half_x * y * y)
    return y


def _bcast(s):
    return lax.broadcast_in_dim(s, (16,), ())


def _ln_rows16(rows, tbuf, base, wv, bv, lane):
    inv_h = jnp.float32(1.0 / HIDDEN)
    for rr in range(16):
        r = base + rr
        v = [rows[r, pl.ds(k * 16, 16)] for k in range(HIDDEN // 16)]
        s = (v[0] + v[1]) + (v[2] + v[3])
        q = (v[0] * v[0] + v[1] * v[1]) + (v[2] * v[2] + v[3] * v[3])
        mean = _bcast(jnp.sum(s)) * inv_h
        var = _bcast(jnp.sum(q)) * inv_h - mean * mean
        inv = _rsqrt16(var + EPS)
        rcol = jnp.full((16,), r, jnp.int32)
        for k in range(HIDDEN // 16):
            scale = inv * wv[k]
            o = (v[k] - mean) * scale + bv[k]
            plsc.store_scatter(tbuf, [k * 16 + lane, rcol], o)


def _ln_body(xt_hbm, tbl_hbm, w_hbm, b_hbm, out_hbm, idx_v, rows2, tbuf,
             w_v, b_v, gsems, osem, *, nseq):
    cid = lax.axis_index("c")
    sid = lax.axis_index("s")
    wid = sid * 2 + cid
    i0 = wid * KTOK

    pltpu.sync_copy(xt_hbm.at[:, pl.ds(i0, KTOK)], idx_v)
    pltpu.sync_copy(w_hbm, w_v)
    pltpu.sync_copy(b_hbm, b_v)

    wv = [w_v[pl.ds(k * 16, 16)] for k in range(HIDDEN // 16)]
    bv = [b_v[pl.ds(k * 16, 16)] for k in range(HIDDEN // 16)]
    lane = lax.iota(jnp.int32, 16)

    def gather(j, b):
        pltpu.make_async_copy(
            tbl_hbm.at[idx_v.at[j]], rows2.at[b], gsems.at[b]).start()

    def gather_wait(j, b):
        pltpu.make_async_copy(
            tbl_hbm.at[idx_v.at[j]], rows2.at[b], gsems.at[b]).wait()

    def out_start(j):
        pltpu.make_async_copy(
            tbuf, out_hbm.at[j, :, pl.ds(i0, KTOK)], osem).start()

    def out_wait(j):
        pltpu.make_async_copy(
            tbuf, out_hbm.at[j, :, pl.ds(i0, KTOK)], osem).wait()

    gather(0, 0)
    gather(1, 1)

    def pair_body(p, carry):
        for b in range(2):
            j = 2 * p + b
            gather_wait(j, b)

            @pl.when(j >= 1)
            def _():
                out_wait(j - 1)

            def grp_body(g, c2):
                _ln_rows16(rows2.at[b], tbuf, g * 16, wv, bv, lane)
                return c2

            lax.fori_loop(0, KTOK // 16, grp_body, 0)
            out_start(j)

            @pl.when(j + 2 < nseq)
            def _():
                gather(j + 2, b)
        return carry

    lax.fori_loop(0, nseq // 2, pair_body, 0)
    out_wait(nseq - 1)


def kernel(x, word_table, ln_weight, ln_bias):
    rows, cols = x.shape
    xt = x.T.astype(jnp.int32)

    mesh = plsc.VectorSubcoreMesh(core_axis_name="c", subcore_axis_name="s")
    run = pl.kernel(
        functools.partial(_ln_body, nseq=cols),
        mesh=mesh,
        compiler_params=pltpu.CompilerParams(
            needs_layout_passes=False, use_tc_tiling_on_sc=False),
        out_type=jax.ShapeDtypeStruct((cols, HIDDEN, rows), jnp.float32),
        scratch_types=[
            pltpu.VMEM((cols, KTOK), jnp.int32),
            pltpu.VMEM((2, KTOK, HIDDEN), jnp.float32),
            pltpu.VMEM((HIDDEN, KTOK), jnp.float32),
            pltpu.VMEM((HIDDEN,), jnp.float32),
            pltpu.VMEM((HIDDEN,), jnp.float32),
            pltpu.SemaphoreType.DMA((2,)),
            pltpu.SemaphoreType.DMA,
        ],
    )
    out = run(xt, word_table, ln_weight, ln_bias)
    return out.transpose(2, 0, 1)


# speedup vs baseline: 1.6199x; 1.6199x over previous
import functools

import jax
import jax.numpy as jnp
from jax import lax
from jax.experimental import pallas as pl
from jax.experimental.pallas import tpu as pltpu
from jax.experimental.pallas import tpu_sc as plsc

VOCAB = 1000000
HIDDEN = 64
EPS = 1e-5

NW = 32
CHUNK = 128


def _rsqrt16(x):
    xi = plsc.bitcast(x, jnp.int32)
    yi = jnp.int32(0x5F3759DF) - (xi >> 1)
    y = plsc.bitcast(yi, jnp.float32)
    half_x = 0.5 * x
    for _ in range(3):
        y = y * (1.5 - half_x * y * y)
    return y


def _bcast(s):
    return lax.broadcast_in_dim(s, (16,), ())


def _ln_rows16(rows, obuf, base, wv, bv):
    inv_h = jnp.float32(1.0 / HIDDEN)
    for rr in range(16):
        r = base + rr
        v = [rows[r, pl.ds(k * 16, 16)] for k in range(HIDDEN // 16)]
        s = (v[0] + v[1]) + (v[2] + v[3])
        q = (v[0] * v[0] + v[1] * v[1]) + (v[2] * v[2] + v[3] * v[3])
        mean = _bcast(jnp.sum(s)) * inv_h
        var = _bcast(jnp.sum(q)) * inv_h - mean * mean
        inv = _rsqrt16(var + EPS)
        for k in range(HIDDEN // 16):
            scale = inv * wv[k]
            obuf[r, pl.ds(k * 16, 16)] = (v[k] - mean) * scale + bv[k]


def _ln_body(x_hbm, tbl_hbm, w_hbm, b_hbm, out_hbm, idx_v, rows2, obuf2,
             w_v, b_v, gsems, osems, *, nchunk):
    cid = lax.axis_index("c")
    sid = lax.axis_index("s")
    wid = sid * 2 + cid

    pltpu.sync_copy(x_hbm.at[wid], idx_v)
    pltpu.sync_copy(w_hbm, w_v)
    pltpu.sync_copy(b_hbm, b_v)

    wv = [w_v[pl.ds(k * 16, 16)] for k in range(HIDDEN // 16)]
    bv = [b_v[pl.ds(k * 16, 16)] for k in range(HIDDEN // 16)]

    def gather(i, b):
        pltpu.make_async_copy(
            tbl_hbm.at[idx_v.at[i]], rows2.at[b], gsems.at[b]).start()

    def gather_wait(i, b):
        pltpu.make_async_copy(
            tbl_hbm.at[idx_v.at[i]], rows2.at[b], gsems.at[b]).wait()

    def out_start(i, b):
        pltpu.make_async_copy(
            obuf2.at[b], out_hbm.at[wid, i], osems.at[b]).start()

    def out_wait(i, b):
        pltpu.make_async_copy(
            obuf2.at[b], out_hbm.at[wid, i], osems.at[b]).wait()

    gather(0, 0)
    gather(1, 1)

    def pair_body(p, carry):
        for b in range(2):
            i = 2 * p + b
            gather_wait(i, b)

            @pl.when(i >= 2)
            def _():
                out_wait(i - 2, b)

            def grp_body(g, c2):
                _ln_rows16(rows2.at[b], obuf2.at[b], g * 16, wv, bv)
                return c2

            lax.fori_loop(0, CHUNK // 16, grp_body, 0)
            out_start(i, b)

            @pl.when(i + 2 < nchunk)
            def _():
                gather(i + 2, b)
        return carry

    lax.fori_loop(0, nchunk // 2, pair_body, 0)
    out_wait(nchunk - 2, 0)
    out_wait(nchunk - 1, 1)


def kernel(x, word_table, ln_weight, ln_bias):
    rows, cols = x.shape
    total = rows * cols
    nchunk = total // (NW * CHUNK)
    xf = x.reshape(NW, nchunk, CHUNK).astype(jnp.int32)

    mesh = plsc.VectorSubcoreMesh(core_axis_name="c", subcore_axis_name="s")
    run = pl.kernel(
        functools.partial(_ln_body, nchunk=nchunk),
        mesh=mesh,
        compiler_params=pltpu.CompilerParams(
            needs_layout_passes=False, use_tc_tiling_on_sc=False),
        out_type=jax.ShapeDtypeStruct((NW, nchunk, CHUNK, HIDDEN), jnp.float32),
        scratch_types=[
            pltpu.VMEM((nchunk, CHUNK), jnp.int32),
            pltpu.VMEM((2, CHUNK, HIDDEN), jnp.float32),
            pltpu.VMEM((2, CHUNK, HIDDEN), jnp.float32),
            pltpu.VMEM((HIDDEN,), jnp.float32),
            pltpu.VMEM((HIDDEN,), jnp.float32),
            pltpu.SemaphoreType.DMA((2,)),
            pltpu.SemaphoreType.DMA((2,)),
        ],
    )
    out = run(xf, word_table, ln_weight, ln_bias)
    return out.reshape(rows, cols, HIDDEN)


# speedup vs baseline: 1.8089x; 1.1167x over previous
import functools

import jax
import jax.numpy as jnp
from jax import lax
from jax.experimental import pallas as pl
from jax.experimental.pallas import tpu as pltpu
from jax.experimental.pallas import tpu_sc as plsc

VOCAB = 1000000
HIDDEN = 64
EPS = 1e-5

NW = 32
CHUNK = 128


def _rsqrt16(x):
    xi = plsc.bitcast(x, jnp.int32)
    yi = jnp.int32(0x5F3759DF) - (xi >> 1)
    y = plsc.bitcast(yi, jnp.float32)
    half_x = 0.5 * x
    for _ in range(3):
        y = y * (1.5 - half_x * y * y)
    return y


def _bcast(s):
    return lax.broadcast_in_dim(s, (16,), ())


def _ln_rows16(rows, base, wv, bv):
    inv_h = jnp.float32(1.0 / HIDDEN)
    for rr in range(16):
        r = base + rr
        v = [rows[r, pl.ds(k * 16, 16)] for k in range(HIDDEN // 16)]
        s = (v[0] + v[1]) + (v[2] + v[3])
        q = (v[0] * v[0] + v[1] * v[1]) + (v[2] * v[2] + v[3] * v[3])
        mean = _bcast(jnp.sum(s)) * inv_h
        var = _bcast(jnp.sum(q)) * inv_h - mean * mean
        inv = _rsqrt16(var + EPS)
        for k in range(HIDDEN // 16):
            scale = inv * wv[k]
            rows[r, pl.ds(k * 16, 16)] = (v[k] - mean) * scale + bv[k]


def _ln_body(x_hbm, tbl_hbm, w_hbm, b_hbm, out_hbm, idx_v, rows2, w_v, b_v,
             sems, *, nchunk):
    cid = lax.axis_index("c")
    sid = lax.axis_index("s")
    wid = sid * 2 + cid

    pltpu.sync_copy(x_hbm.at[wid], idx_v)
    pltpu.sync_copy(w_hbm, w_v)
    pltpu.sync_copy(b_hbm, b_v)

    wv = [w_v[pl.ds(k * 16, 16)] for k in range(HIDDEN // 16)]
    bv = [b_v[pl.ds(k * 16, 16)] for k in range(HIDDEN // 16)]

    def gather(i, b):
        pltpu.make_async_copy(
            tbl_hbm.at[idx_v.at[i]], rows2.at[b], sems.at[b]).start()

    def gather_wait(i, b):
        pltpu.make_async_copy(
            tbl_hbm.at[idx_v.at[i]], rows2.at[b], sems.at[b]).wait()

    gather(0, 0)
    gather(1, 1)

    def pair_body(p, carry):
        for b in range(2):
            i = 2 * p + b
            gather_wait(i, b)

            def grp_body(g, c2):
                _ln_rows16(rows2.at[b], g * 16, wv, bv)
                return c2

            lax.fori_loop(0, CHUNK // 16, grp_body, 0)
            pltpu.sync_copy(rows2.at[b], out_hbm.at[wid, i])

            @pl.when(i + 2 < nchunk)
            def _():
                gather(i + 2, b)
        return carry

    lax.fori_loop(0, nchunk // 2, pair_body, 0)


def kernel(x, word_table, ln_weight, ln_bias):
    rows, cols = x.shape
    total = rows * cols
    nchunk = total // (NW * CHUNK)
    xf = x.reshape(NW, nchunk, CHUNK).astype(jnp.int32)

    mesh = plsc.VectorSubcoreMesh(core_axis_name="c", subcore_axis_name="s")
    run = pl.kernel(
        functools.partial(_ln_body, nchunk=nchunk),
        mesh=mesh,
        compiler_params=pltpu.CompilerParams(
            needs_layout_passes=False, use_tc_tiling_on_sc=False),
        out_type=jax.ShapeDtypeStruct((NW, nchunk, CHUNK, HIDDEN), jnp.float32),
        scratch_types=[
            pltpu.VMEM((nchunk, CHUNK), jnp.int32),
            pltpu.VMEM((2, CHUNK, HIDDEN), jnp.float32),
            pltpu.VMEM((HIDDEN,), jnp.float32),
            pltpu.VMEM((HIDDEN,), jnp.float32),
            pltpu.SemaphoreType.DMA((2,)),
        ],
    )
    out = run(xf, word_table, ln_weight, ln_bias)
    return out.reshape(rows, cols, HIDDEN)


# speedup vs baseline: 1.8300x; 1.0117x over previous
import functools

import jax
import jax.numpy as jnp
from jax import lax
from jax.experimental import pallas as pl
from jax.experimental.pallas import tpu as pltpu
from jax.experimental.pallas import tpu_sc as plsc

VOCAB = 1000000
HIDDEN = 64
EPS = 1e-5

NW = 32
CHUNK = 128


def _rsqrt16(x):
    xi = plsc.bitcast(x, jnp.int32)
    yi = jnp.int32(0x5F3759DF) - (xi >> 1)
    y = plsc.bitcast(yi, jnp.float32)
    half_x = 0.5 * x
    for _ in range(2):
        y = y * (1.5 - half_x * y * y)
    return y


def _bcast(s):
    return lax.broadcast_in_dim(s, (16,), ())


def _ln_rows16(rows, base, wv, bv):
    inv_h = jnp.float32(1.0 / HIDDEN)
    for rr in range(16):
        r = base + rr
        v = [rows[r, pl.ds(k * 16, 16)] for k in range(HIDDEN // 16)]
        s = (v[0] + v[1]) + (v[2] + v[3])
        q = (v[0] * v[0] + v[1] * v[1]) + (v[2] * v[2] + v[3] * v[3])
        mean = _bcast(jnp.sum(s)) * inv_h
        var = _bcast(jnp.sum(q)) * inv_h - mean * mean
        inv = _rsqrt16(var + EPS)
        for k in range(HIDDEN // 16):
            scale = inv * wv[k]
            rows[r, pl.ds(k * 16, 16)] = (v[k] - mean) * scale + bv[k]


def _ln_body(x_hbm, tbl_hbm, w_hbm, b_hbm, out_hbm, idx_v, rows2, w_v, b_v,
             sems, *, nchunk):
    cid = lax.axis_index("c")
    sid = lax.axis_index("s")
    wid = sid * 2 + cid

    pltpu.sync_copy(x_hbm.at[wid], idx_v)
    pltpu.sync_copy(w_hbm, w_v)
    pltpu.sync_copy(b_hbm, b_v)

    wv = [w_v[pl.ds(k * 16, 16)] for k in range(HIDDEN // 16)]
    bv = [b_v[pl.ds(k * 16, 16)] for k in range(HIDDEN // 16)]

    def gather(i, b):
        pltpu.make_async_copy(
            tbl_hbm.at[idx_v.at[i]], rows2.at[b], sems.at[b]).start()

    def gather_wait(i, b):
        pltpu.make_async_copy(
            tbl_hbm.at[idx_v.at[i]], rows2.at[b], sems.at[b]).wait()

    for b in range(4):
        gather(b, b)

    def quad_body(p, carry):
        for h in range(2):
            base = 4 * p + 2 * h
            for b in range(2):
                i = base + b
                gather_wait(i, 2 * h + b)

                def grp_body(g, c2, _b=2 * h + b):
                    _ln_rows16(rows2.at[_b], g * 16, wv, bv)
                    return c2

                lax.fori_loop(0, CHUNK // 16, grp_body, 0)
            pltpu.sync_copy(
                rows2.at[pl.ds(2 * h, 2)], out_hbm.at[wid, pl.ds(base, 2)])

            @pl.when(base + 4 < nchunk)
            def _():
                gather(base + 4, 2 * h)
                gather(base + 5, 2 * h + 1)
        return carry

    lax.fori_loop(0, nchunk // 4, quad_body, 0)


def kernel(x, word_table, ln_weight, ln_bias):
    rows, cols = x.shape
    total = rows * cols
    nchunk = total // (NW * CHUNK)
    xf = x.reshape(NW, nchunk, CHUNK).astype(jnp.int32)

    mesh = plsc.VectorSubcoreMesh(core_axis_name="c", subcore_axis_name="s")
    run = pl.kernel(
        functools.partial(_ln_body, nchunk=nchunk),
        mesh=mesh,
        compiler_params=pltpu.CompilerParams(
            needs_layout_passes=False, use_tc_tiling_on_sc=False),
        out_type=jax.ShapeDtypeStruct((NW, nchunk, CHUNK, HIDDEN), jnp.float32),
        scratch_types=[
            pltpu.VMEM((nchunk, CHUNK), jnp.int32),
            pltpu.VMEM((4, CHUNK, HIDDEN), jnp.float32),
            pltpu.VMEM((HIDDEN,), jnp.float32),
            pltpu.VMEM((HIDDEN,), jnp.float32),
            pltpu.SemaphoreType.DMA((4,)),
        ],
    )
    out = run(xf, word_table, ln_weight, ln_bias)
    return out.reshape(rows, cols, HIDDEN)
